# Initial kernel scaffold; baseline (speedup 1.0000x reference)
import jax
import jax.numpy as jnp
from jax.experimental import pallas as pl


def kernel(x, edge_index, edge_weight, Wxz, bxz, Whz, bhz, Wxr, bxr, Whr, bhr, Wxh, bxh, Whh, bhh, W1, b1, W2, b2, W3, b3):
    raise NotImplementedError("write your pallas kernel here")


# speedup vs baseline: 8.4259x; 8.4259x over previous
import functools

import jax
import jax.numpy as jnp
from jax import lax
from jax.experimental import pallas as pl
from jax.experimental.pallas import tpu as pltpu
from jax.experimental.pallas import tpu_sc as plsc

N = 10000
F = 128
E = 320000
NUM_CLASSES = 5

NW = 32
CHUNK = 128
CPW = 80
EPW = CPW * CHUNK
E_PAD = NW * EPW
RE = E_PAD // CHUNK
NP = 10112
NPS = NP // 16

_mesh = plsc.VectorSubcoreMesh(core_axis_name="c", subcore_axis_name="s")
_sc_params = pltpu.CompilerParams(needs_layout_passes=False)


@functools.partial(
    pl.kernel,
    out_type=jax.ShapeDtypeStruct((2 * NP,), jnp.float32),
    mesh=_mesh,
    compiler_params=_sc_params,
    scratch_types=[
        pltpu.VMEM((EPW,), jnp.int32),
        pltpu.VMEM((EPW,), jnp.int32),
        pltpu.VMEM((EPW,), jnp.float32),
        pltpu.VMEM((CPW, CHUNK), jnp.int32),
        pltpu.VMEM((NPS,), jnp.float32),
        pltpu.VMEM_SHARED((NP,), jnp.float32),
        pltpu.SemaphoreType.DMA,
    ],
)
def _deg(src1_hbm, dst1_hbm, ew1_hbm, src2_hbm, z1_hbm, deg_out,
         sflat, dflat, wflat, srcv2, zbuf, acc, sem):
    c = lax.axis_index("c")
    s = lax.axis_index("s")
    wid = s * 2 + c
    base1 = wid * EPW
    pltpu.sync_copy(src1_hbm.at[pl.ds(base1, EPW)], sflat)
    pltpu.sync_copy(dst1_hbm.at[pl.ds(base1, EPW)], dflat)
    pltpu.sync_copy(ew1_hbm.at[pl.ds(base1, EPW)], wflat)
    pltpu.sync_copy(src2_hbm.at[pl.ds(wid * CPW, CPW)], srcv2)
    pltpu.sync_copy(z1_hbm.at[pl.ds(s * NPS, NPS)], zbuf)
    pltpu.sync_copy(zbuf, acc.at[pl.ds(s * NPS, NPS)])

    def mask_body(i, carry):
        sl = pl.ds(i * 16, 16)
        wflat[sl] = jnp.where(sflat[sl] != dflat[sl], wflat[sl], 0.0)
        return carry

    lax.fori_loop(0, EPW // 16, mask_body, 0)
    plsc.subcore_barrier()

    def chunk_body(j, carry):
        pltpu.sync_copy(wflat.at[pl.ds(j * CHUNK, CHUNK)],
                        acc.at[srcv2.at[j]], add=True)
        return carry

    lax.fori_loop(0, CPW, chunk_body, 0)
    plsc.subcore_barrier()
    pltpu.sync_copy(acc.at[pl.ds(s * NPS, NPS)], zbuf)
    pltpu.sync_copy(zbuf, deg_out.at[pl.ds(c * NP + s * NPS, NPS)])


G = 8
NG = CPW // G
GE = G * CHUNK


@functools.partial(
    pl.kernel,
    out_type=jax.ShapeDtypeStruct((2, NP, F), jnp.float32),
    mesh=_mesh,
    compiler_params=_sc_params,
    scratch_types=[
        pltpu.VMEM((GE,), jnp.int32),
        pltpu.VMEM((GE,), jnp.int32),
        pltpu.VMEM((GE,), jnp.float32),
        pltpu.VMEM((NP,), jnp.float32),
        pltpu.VMEM((G, CHUNK), jnp.int32),
        pltpu.VMEM((G, CHUNK), jnp.int32),
        pltpu.VMEM((CHUNK, F), jnp.float32),
        pltpu.VMEM_SHARED((NP, F), jnp.float32),
        pltpu.SemaphoreType.DMA,
    ],
)
def _row_pass(tab_hbm, src1_hbm, dst1_hbm, ew1_hbm, src2_hbm, dst2_hbm,
              dis_hbm, znf_hbm, s_out,
              sflat, dflat, wflat, disv, srcv2, dstv2, rows, acc, sem):
    c = lax.axis_index("c")
    s = lax.axis_index("s")
    wid = s * 2 + c
    pltpu.sync_copy(dis_hbm.at[pl.ds(0, NP)], disv)
    pltpu.sync_copy(znf_hbm.at[pl.ds(s * NPS, NPS)], acc.at[pl.ds(s * NPS, NPS)])
    plsc.subcore_barrier()

    def group_body(m, carry):
        base1 = wid * EPW + m * GE
        base2 = wid * CPW + m * G
        pltpu.sync_copy(src1_hbm.at[pl.ds(base1, GE)], sflat)
        pltpu.sync_copy(dst1_hbm.at[pl.ds(base1, GE)], dflat)
        pltpu.sync_copy(ew1_hbm.at[pl.ds(base1, GE)], wflat)
        pltpu.sync_copy(src2_hbm.at[pl.ds(base2, G)], srcv2)
        pltpu.sync_copy(dst2_hbm.at[pl.ds(base2, G)], dstv2)

        def scale_body(i, carry2):
            sl = pl.ds(i * 16, 16)
            s16 = sflat[sl]
            d16 = dflat[sl]
            w = plsc.load_gather(disv, [s16]) * wflat[sl] * plsc.load_gather(disv, [d16])
            wflat[sl] = jnp.where(s16 != d16, -w, 0.0)
            return carry2

        lax.fori_loop(0, GE // 16, scale_body, 0)

        def chunk_body(j, carry2):
            pltpu.async_copy(tab_hbm.at[srcv2.at[j]], rows, sem).wait()

            def row_body(r, carry3):
                idx = jnp.full((16,), j * CHUNK + r, jnp.int32)
                w = plsc.load_gather(wflat, [idx])
                for g in range(F // 16):
                    sl = pl.ds(g * 16, 16)
                    rows[r, sl] = rows[r, sl] * w
                return carry3

            lax.fori_loop(0, CHUNK, row_body, 0)
            pltpu.sync_copy(rows, acc.at[dstv2.at[j]], add=True)
            return carry2

        lax.fori_loop(0, G, chunk_body, 0)
        return carry

    lax.fori_loop(0, NG, group_body, 0)
    plsc.subcore_barrier()
    pltpu.sync_copy(acc.at[pl.ds(s * NPS, NPS)],
                    s_out.at[c, pl.ds(s * NPS, NPS)])


_BLK = NPS


def _dis_body(degp_ref, dis_ref):
    deg = degp_ref[0:1, :] + degp_ref[1:2, :]
    dis_ref[...] = jnp.where(deg > 0, lax.rsqrt(deg), 0.0)


_dis_tc = pl.pallas_call(
    _dis_body,
    out_shape=jax.ShapeDtypeStruct((1, NP), jnp.float32),
)


def _mid_body(s1_ref, tx1_ref):
    tx1_ref[...] = s1_ref[0] + s1_ref[1]


_mid = pl.pallas_call(
    _mid_body,
    grid=(NP // _BLK,),
    in_specs=[pl.BlockSpec((2, _BLK, F), lambda i: (0, i, 0))],
    out_specs=pl.BlockSpec((_BLK, F), lambda i: (i, 0)),
    out_shape=jax.ShapeDtypeStruct((NP, F), jnp.float32),
)


def _final_body(x_ref, tx1_ref, s2_ref, wz_ref, wh_ref, bz_ref,
                bh_ref, w1_ref, b1_ref, w2_ref, b2_ref, w3_ref, b3_ref,
                out_ref):
    x = x_ref[...]
    tx1 = tx1_ref[...]
    tx2 = 2.0 * (s2_ref[0] + s2_ref[1]) - x
    xcat = jnp.concatenate([x, tx1, tx2], axis=1)
    cz = jnp.dot(xcat, wz_ref[...], preferred_element_type=jnp.float32) + bz_ref[...]
    ch = jnp.dot(xcat, wh_ref[...], preferred_element_type=jnp.float32) + bh_ref[...]
    z = jax.nn.sigmoid(cz)
    h = (1.0 - z) * jnp.tanh(ch)
    h = jnp.maximum(h, 0.0)
    h = jnp.maximum(jnp.dot(h, w1_ref[...], preferred_element_type=jnp.float32) + b1_ref[...], 0.0)
    h = jnp.maximum(jnp.dot(h, w2_ref[...], preferred_element_type=jnp.float32) + b2_ref[...], 0.0)
    lg = jnp.dot(h, w3_ref[...], preferred_element_type=jnp.float32) + b3_ref[...]
    col = lax.broadcasted_iota(jnp.int32, lg.shape, 1)
    valid = col < NUM_CLASSES
    m = jnp.max(jnp.where(valid, lg, -1e30), axis=1, keepdims=True)
    ssum = jnp.sum(jnp.where(valid, jnp.exp(lg - m), 0.0), axis=1, keepdims=True)
    res = lg - m - jnp.log(ssum)
    out_ref[...] = res[:, :NUM_CLASSES]


def _full(shape):
    return pl.BlockSpec(shape, lambda i: (0,) * len(shape))


_final = pl.pallas_call(
    _final_body,
    grid=(NP // _BLK,),
    in_specs=[
        pl.BlockSpec((_BLK, F), lambda i: (i, 0)),
        pl.BlockSpec((_BLK, F), lambda i: (i, 0)),
        pl.BlockSpec((2, _BLK, F), lambda i: (0, i, 0)),
        _full((3 * F, F)),
        _full((3 * F, F)),
        _full((1, F)),
        _full((1, F)),
        _full((F, 32)),
        _full((1, 32)),
        _full((32, 16)),
        _full((1, 16)),
        _full((16, F)),
        _full((1, F)),
    ],
    out_specs=pl.BlockSpec((_BLK, NUM_CLASSES), lambda i: (i, 0)),
    out_shape=jax.ShapeDtypeStruct((NP, NUM_CLASSES), jnp.float32),
)


def kernel(x, edge_index, edge_weight, Wxz, bxz, Whz, bhz, Wxr, bxr, Whr, bhr,
           Wxh, bxh, Whh, bhh, W1, b1, W2, b2, W3, b3):
    pad = E_PAD - E
    src1 = jnp.concatenate([edge_index[0], jnp.zeros((pad,), jnp.int32)])
    dst1 = jnp.concatenate([edge_index[1], jnp.zeros((pad,), jnp.int32)])
    ew1 = jnp.concatenate([edge_weight, jnp.zeros((pad,), jnp.float32)])
    src2 = src1.reshape(RE, CHUNK)
    dst2 = dst1.reshape(RE, CHUNK)
    z1 = jnp.zeros((NP,), jnp.float32)
    znf = jnp.zeros((NP, F), jnp.float32)
    xp = jnp.concatenate([x, jnp.zeros((NP - N, F), jnp.float32)])

    degp = _deg(src1, dst1, ew1, src2, z1)
    dis = _dis_tc(degp.reshape(2, NP)).reshape(NP)
    s1p = _row_pass(xp, src1, dst1, ew1, src2, dst2, dis, znf)
    tx1 = _mid(s1p)
    s2p = _row_pass(tx1, src1, dst1, ew1, src2, dst2, dis, znf)

    bz = (bxz + bhz).reshape(1, F)
    bh = (bxh + bhh).reshape(1, F)
    wz = Wxz.reshape(3 * F, F)
    wh = Wxh.reshape(3 * F, F)
    w3p = jnp.zeros((16, F), jnp.float32).at[:, :NUM_CLASSES].set(W3)
    b3p = jnp.zeros((1, F), jnp.float32).at[0, :NUM_CLASSES].set(b3)

    out = _final(xp, tx1, s2p, wz, wh, bz, bh,
                 W1, b1.reshape(1, 32), W2, b2.reshape(1, 16), w3p, b3p)
    return out[:N]


# speedup vs baseline: 10.1030x; 1.1990x over previous
import functools

import jax
import jax.numpy as jnp
from jax import lax
from jax.experimental import pallas as pl
from jax.experimental.pallas import tpu as pltpu
from jax.experimental.pallas import tpu_sc as plsc

N = 10000
F = 128
E = 320000
NUM_CLASSES = 5

NW = 32
CHUNK = 128
CPW = 80
EPW = CPW * CHUNK
E_PAD = NW * EPW
RE = E_PAD // CHUNK
NP = 10112
NPS = NP // 16

_mesh = plsc.VectorSubcoreMesh(core_axis_name="c", subcore_axis_name="s")
_sc_params = pltpu.CompilerParams(needs_layout_passes=False)


@functools.partial(
    pl.kernel,
    out_type=jax.ShapeDtypeStruct((2 * NP,), jnp.float32),
    mesh=_mesh,
    compiler_params=_sc_params,
    scratch_types=[
        pltpu.VMEM((EPW,), jnp.int32),
        pltpu.VMEM((EPW,), jnp.int32),
        pltpu.VMEM((EPW,), jnp.float32),
        pltpu.VMEM((CPW, CHUNK), jnp.int32),
        pltpu.VMEM((NPS,), jnp.float32),
        pltpu.VMEM_SHARED((NP,), jnp.float32),
        pltpu.SemaphoreType.DMA,
    ],
)
def _deg(src1_hbm, dst1_hbm, ew1_hbm, src2_hbm, z1_hbm, deg_out,
         sflat, dflat, wflat, srcv2, zbuf, acc, sem):
    c = lax.axis_index("c")
    s = lax.axis_index("s")
    wid = s * 2 + c
    base1 = wid * EPW
    pltpu.sync_copy(src1_hbm.at[pl.ds(base1, EPW)], sflat)
    pltpu.sync_copy(dst1_hbm.at[pl.ds(base1, EPW)], dflat)
    pltpu.sync_copy(ew1_hbm.at[pl.ds(base1, EPW)], wflat)
    pltpu.sync_copy(src2_hbm.at[pl.ds(wid * CPW, CPW)], srcv2)
    pltpu.sync_copy(z1_hbm.at[pl.ds(s * NPS, NPS)], zbuf)
    pltpu.sync_copy(zbuf, acc.at[pl.ds(s * NPS, NPS)])

    def mask_body(i, carry):
        sl = pl.ds(i * 16, 16)
        wflat[sl] = jnp.where(sflat[sl] != dflat[sl], wflat[sl], 0.0)
        return carry

    lax.fori_loop(0, EPW // 16, mask_body, 0)
    plsc.subcore_barrier()

    def chunk_body(j, carry):
        pltpu.sync_copy(wflat.at[pl.ds(j * CHUNK, CHUNK)],
                        acc.at[srcv2.at[j]], add=True)
        return carry

    lax.fori_loop(0, CPW, chunk_body, 0)
    plsc.subcore_barrier()
    pltpu.sync_copy(acc.at[pl.ds(s * NPS, NPS)], zbuf)
    pltpu.sync_copy(zbuf, deg_out.at[pl.ds(c * NP + s * NPS, NPS)])


G = 8
NG = CPW // G
GE = G * CHUNK


@functools.partial(
    pl.kernel,
    out_type=jax.ShapeDtypeStruct((2, NP, F), jnp.float32),
    mesh=_mesh,
    compiler_params=_sc_params,
    scratch_types=[
        pltpu.VMEM((GE,), jnp.int32),
        pltpu.VMEM((GE,), jnp.int32),
        pltpu.VMEM((GE,), jnp.float32),
        pltpu.VMEM((NP,), jnp.float32),
        pltpu.VMEM((G, CHUNK), jnp.int32),
        pltpu.VMEM((G, CHUNK), jnp.int32),
        pltpu.VMEM((CHUNK, F), jnp.float32),
        pltpu.VMEM((CHUNK, F), jnp.float32),
        pltpu.VMEM_SHARED((NP, F), jnp.float32),
        pltpu.SemaphoreType.DMA,
        pltpu.SemaphoreType.DMA,
    ],
)
def _row_pass(tab_hbm, src1_hbm, dst1_hbm, ew1_hbm, src2_hbm, dst2_hbm,
              dis_hbm, znf_hbm, s_out,
              sflat, dflat, wflat, disv, srcv2, dstv2, rows_a, rows_b,
              acc, sem_a, sem_b):
    c = lax.axis_index("c")
    s = lax.axis_index("s")
    wid = s * 2 + c
    pltpu.sync_copy(dis_hbm.at[pl.ds(0, NP)], disv)
    pltpu.sync_copy(znf_hbm.at[pl.ds(s * NPS, NPS)], acc.at[pl.ds(s * NPS, NPS)])
    plsc.subcore_barrier()

    def group_body(m, carry):
        base1 = wid * EPW + m * GE
        base2 = wid * CPW + m * G
        pltpu.sync_copy(src1_hbm.at[pl.ds(base1, GE)], sflat)
        pltpu.sync_copy(dst1_hbm.at[pl.ds(base1, GE)], dflat)
        pltpu.sync_copy(ew1_hbm.at[pl.ds(base1, GE)], wflat)
        pltpu.sync_copy(src2_hbm.at[pl.ds(base2, G)], srcv2)
        pltpu.sync_copy(dst2_hbm.at[pl.ds(base2, G)], dstv2)

        def scale_body(i, carry2):
            sl = pl.ds(i * 16, 16)
            s16 = sflat[sl]
            d16 = dflat[sl]
            w = plsc.load_gather(disv, [s16]) * wflat[sl] * plsc.load_gather(disv, [d16])
            wflat[sl] = jnp.where(s16 != d16, -w, 0.0)
            return carry2

        lax.fori_loop(0, GE // 16, scale_body, 0)

        def process(j, rows):
            def row_body(jr, carry3):
                for rr in range(4):
                    r = jr * 4 + rr
                    idx = jnp.full((16,), j * CHUNK + r, jnp.int32)
                    w = plsc.load_gather(wflat, [idx])
                    for g in range(F // 16):
                        sl = pl.ds(g * 16, 16)
                        rows[r, sl] = rows[r, sl] * w
                return carry3

            lax.fori_loop(0, CHUNK // 4, row_body, 0)
            pltpu.sync_copy(rows, acc.at[dstv2.at[j]], add=True)

        pltpu.async_copy(tab_hbm.at[srcv2.at[0]], rows_a, sem_a)

        def pair_body(jj, carry2):
            j = jj * 2
            pltpu.async_copy(tab_hbm.at[srcv2.at[j + 1]], rows_b, sem_b)
            pltpu.make_async_copy(tab_hbm.at[srcv2.at[j]], rows_a, sem_a).wait()
            process(j, rows_a)

            @pl.when(jj + 1 < G // 2)
            def _prefetch():
                pltpu.async_copy(tab_hbm.at[srcv2.at[j + 2]], rows_a, sem_a)

            pltpu.make_async_copy(tab_hbm.at[srcv2.at[j + 1]], rows_b, sem_b).wait()
            process(j + 1, rows_b)
            return carry2

        lax.fori_loop(0, G // 2, pair_body, 0)
        return carry

    lax.fori_loop(0, NG, group_body, 0)
    plsc.subcore_barrier()
    pltpu.sync_copy(acc.at[pl.ds(s * NPS, NPS)],
                    s_out.at[c, pl.ds(s * NPS, NPS)])


_BLK = NPS


def _dis_body(degp_ref, dis_ref):
    deg = degp_ref[0:1, :] + degp_ref[1:2, :]
    dis_ref[...] = jnp.where(deg > 0, lax.rsqrt(deg), 0.0)


_dis_tc = pl.pallas_call(
    _dis_body,
    out_shape=jax.ShapeDtypeStruct((1, NP), jnp.float32),
)


def _mid_body(s1_ref, tx1_ref):
    tx1_ref[...] = s1_ref[0] + s1_ref[1]


_mid = pl.pallas_call(
    _mid_body,
    grid=(NP // _BLK,),
    in_specs=[pl.BlockSpec((2, _BLK, F), lambda i: (0, i, 0))],
    out_specs=pl.BlockSpec((_BLK, F), lambda i: (i, 0)),
    out_shape=jax.ShapeDtypeStruct((NP, F), jnp.float32),
)


def _final_body(x_ref, tx1_ref, s2_ref, wz_ref, wh_ref, bz_ref,
                bh_ref, w1_ref, b1_ref, w2_ref, b2_ref, w3_ref, b3_ref,
                out_ref):
    x = x_ref[...]
    tx1 = tx1_ref[...]
    tx2 = 2.0 * (s2_ref[0] + s2_ref[1]) - x
    xcat = jnp.concatenate([x, tx1, tx2], axis=1)
    cz = jnp.dot(xcat, wz_ref[...], preferred_element_type=jnp.float32) + bz_ref[...]
    ch = jnp.dot(xcat, wh_ref[...], preferred_element_type=jnp.float32) + bh_ref[...]
    z = jax.nn.sigmoid(cz)
    h = (1.0 - z) * jnp.tanh(ch)
    h = jnp.maximum(h, 0.0)
    h = jnp.maximum(jnp.dot(h, w1_ref[...], preferred_element_type=jnp.float32) + b1_ref[...], 0.0)
    h = jnp.maximum(jnp.dot(h, w2_ref[...], preferred_element_type=jnp.float32) + b2_ref[...], 0.0)
    lg = jnp.dot(h, w3_ref[...], preferred_element_type=jnp.float32) + b3_ref[...]
    col = lax.broadcasted_iota(jnp.int32, lg.shape, 1)
    valid = col < NUM_CLASSES
    m = jnp.max(jnp.where(valid, lg, -1e30), axis=1, keepdims=True)
    ssum = jnp.sum(jnp.where(valid, jnp.exp(lg - m), 0.0), axis=1, keepdims=True)
    res = lg - m - jnp.log(ssum)
    out_ref[...] = res[:, :NUM_CLASSES]


def _full(shape):
    return pl.BlockSpec(shape, lambda i: (0,) * len(shape))


_final = pl.pallas_call(
    _final_body,
    grid=(NP // _BLK,),
    in_specs=[
        pl.BlockSpec((_BLK, F), lambda i: (i, 0)),
        pl.BlockSpec((_BLK, F), lambda i: (i, 0)),
        pl.BlockSpec((2, _BLK, F), lambda i: (0, i, 0)),
        _full((3 * F, F)),
        _full((3 * F, F)),
        _full((1, F)),
        _full((1, F)),
        _full((F, 32)),
        _full((1, 32)),
        _full((32, 16)),
        _full((1, 16)),
        _full((16, F)),
        _full((1, F)),
    ],
    out_specs=pl.BlockSpec((_BLK, NUM_CLASSES), lambda i: (i, 0)),
    out_shape=jax.ShapeDtypeStruct((NP, NUM_CLASSES), jnp.float32),
)


def kernel(x, edge_index, edge_weight, Wxz, bxz, Whz, bhz, Wxr, bxr, Whr, bhr,
           Wxh, bxh, Whh, bhh, W1, b1, W2, b2, W3, b3):
    pad = E_PAD - E
    src1 = jnp.concatenate([edge_index[0], jnp.zeros((pad,), jnp.int32)])
    dst1 = jnp.concatenate([edge_index[1], jnp.zeros((pad,), jnp.int32)])
    ew1 = jnp.concatenate([edge_weight, jnp.zeros((pad,), jnp.float32)])
    src2 = src1.reshape(RE, CHUNK)
    dst2 = dst1.reshape(RE, CHUNK)
    z1 = jnp.zeros((NP,), jnp.float32)
    znf = jnp.zeros((NP, F), jnp.float32)
    xp = jnp.concatenate([x, jnp.zeros((NP - N, F), jnp.float32)])

    degp = _deg(src1, dst1, ew1, src2, z1)
    dis = _dis_tc(degp.reshape(2, NP)).reshape(NP)
    s1p = _row_pass(xp, src1, dst1, ew1, src2, dst2, dis, znf)
    tx1 = _mid(s1p)
    s2p = _row_pass(tx1, src1, dst1, ew1, src2, dst2, dis, znf)

    bz = (bxz + bhz).reshape(1, F)
    bh = (bxh + bhh).reshape(1, F)
    wz = Wxz.reshape(3 * F, F)
    wh = Wxh.reshape(3 * F, F)
    w3p = jnp.zeros((16, F), jnp.float32).at[:, :NUM_CLASSES].set(W3)
    b3p = jnp.zeros((1, F), jnp.float32).at[0, :NUM_CLASSES].set(b3)

    out = _final(xp, tx1, s2p, wz, wh, bz, bh,
                 W1, b1.reshape(1, 32), W2, b2.reshape(1, 16), w3p, b3p)
    return out[:N]


# speedup vs baseline: 12.5732x; 1.2445x over previous
import functools

import jax
import jax.numpy as jnp
from jax import lax
from jax.experimental import pallas as pl
from jax.experimental.pallas import tpu as pltpu
from jax.experimental.pallas import tpu_sc as plsc

N = 10000
F = 128
E = 320000
NUM_CLASSES = 5

NW = 32
CHUNK = 128
CPW = 80
EPW = CPW * CHUNK
E_PAD = NW * EPW
RE = E_PAD // CHUNK
NP = 10112
NPS = NP // 16

_mesh = plsc.VectorSubcoreMesh(core_axis_name="c", subcore_axis_name="s")
_sc_params = pltpu.CompilerParams(needs_layout_passes=False)


@functools.partial(
    pl.kernel,
    out_type=jax.ShapeDtypeStruct((2 * NP,), jnp.float32),
    mesh=_mesh,
    compiler_params=_sc_params,
    scratch_types=[
        pltpu.VMEM((EPW,), jnp.int32),
        pltpu.VMEM((EPW,), jnp.int32),
        pltpu.VMEM((EPW,), jnp.float32),
        pltpu.VMEM((CPW, CHUNK), jnp.int32),
        pltpu.VMEM((NPS,), jnp.float32),
        pltpu.VMEM_SHARED((NP,), jnp.float32),
        pltpu.SemaphoreType.DMA,
    ],
)
def _deg(src1_hbm, dst1_hbm, ew1_hbm, src2_hbm, z1_hbm, deg_out,
         sflat, dflat, wflat, srcv2, zbuf, acc, sem):
    c = lax.axis_index("c")
    s = lax.axis_index("s")
    wid = s * 2 + c
    base1 = wid * EPW
    pltpu.sync_copy(src1_hbm.at[pl.ds(base1, EPW)], sflat)
    pltpu.sync_copy(dst1_hbm.at[pl.ds(base1, EPW)], dflat)
    pltpu.sync_copy(ew1_hbm.at[pl.ds(base1, EPW)], wflat)
    pltpu.sync_copy(src2_hbm.at[pl.ds(wid * CPW, CPW)], srcv2)
    pltpu.sync_copy(z1_hbm.at[pl.ds(s * NPS, NPS)], zbuf)
    pltpu.sync_copy(zbuf, acc.at[pl.ds(s * NPS, NPS)])

    def mask_body(i, carry):
        sl = pl.ds(i * 16, 16)
        wflat[sl] = jnp.where(sflat[sl] != dflat[sl], wflat[sl], 0.0)
        return carry

    lax.fori_loop(0, EPW // 16, mask_body, 0)
    plsc.subcore_barrier()

    def chunk_body(j, carry):
        pltpu.sync_copy(wflat.at[pl.ds(j * CHUNK, CHUNK)],
                        acc.at[srcv2.at[j]], add=True)
        return carry

    lax.fori_loop(0, CPW, chunk_body, 0)
    plsc.subcore_barrier()
    pltpu.sync_copy(acc.at[pl.ds(s * NPS, NPS)], zbuf)
    pltpu.sync_copy(zbuf, deg_out.at[pl.ds(c * NP + s * NPS, NPS)])


G = 8
NG = CPW // G
GE = G * CHUNK
CPW0 = 120
CPW1 = 2 * CPW - CPW0


@functools.partial(
    pl.kernel,
    out_type=jax.ShapeDtypeStruct((2, NP, F), jnp.float32),
    mesh=_mesh,
    compiler_params=_sc_params,
    scratch_types=[
        pltpu.VMEM((GE,), jnp.int32),
        pltpu.VMEM((GE,), jnp.int32),
        pltpu.VMEM((GE,), jnp.float32),
        pltpu.VMEM((NP,), jnp.float32),
        pltpu.VMEM((G, CHUNK), jnp.int32),
        pltpu.VMEM((G, CHUNK), jnp.int32),
        pltpu.VMEM((CHUNK, F), jnp.float32),
        pltpu.VMEM((CHUNK, F), jnp.float32),
        pltpu.VMEM_SHARED((NP, F), jnp.float32),
        pltpu.SemaphoreType.DMA,
        pltpu.SemaphoreType.DMA,
    ],
)
def _row_pass(tab_hbm, src1_hbm, dst1_hbm, ew1_hbm, src2_hbm, dst2_hbm,
              dis_hbm, znf_hbm, s_out,
              sflat, dflat, wflat, disv, srcv2, dstv2, rows_a, rows_b,
              acc, sem_a, sem_b):
    c = lax.axis_index("c")
    s = lax.axis_index("s")
    my_cpw = jnp.where(c == 0, CPW0, CPW1)
    chunk0 = s * (2 * CPW) + c * CPW0
    pltpu.sync_copy(dis_hbm.at[pl.ds(0, NP)], disv)
    pltpu.sync_copy(znf_hbm.at[pl.ds(s * NPS, NPS)], acc.at[pl.ds(s * NPS, NPS)])
    plsc.subcore_barrier()

    def group_body(m, carry):
        base2 = chunk0 + m * G
        base1 = base2 * CHUNK
        pltpu.sync_copy(src1_hbm.at[pl.ds(base1, GE)], sflat)
        pltpu.sync_copy(dst1_hbm.at[pl.ds(base1, GE)], dflat)
        pltpu.sync_copy(ew1_hbm.at[pl.ds(base1, GE)], wflat)
        pltpu.sync_copy(src2_hbm.at[pl.ds(base2, G)], srcv2)
        pltpu.sync_copy(dst2_hbm.at[pl.ds(base2, G)], dstv2)

        def scale_body(i, carry2):
            sl = pl.ds(i * 16, 16)
            s16 = sflat[sl]
            d16 = dflat[sl]
            w = plsc.load_gather(disv, [s16]) * wflat[sl] * plsc.load_gather(disv, [d16])
            wflat[sl] = jnp.where(s16 != d16, -w, 0.0)
            return carry2

        lax.fori_loop(0, GE // 16, scale_body, 0)

        def process(j, rows):
            def row_body(jr, carry3):
                for rr in range(4):
                    r = jr * 4 + rr
                    idx = jnp.full((16,), j * CHUNK + r, jnp.int32)
                    w = plsc.load_gather(wflat, [idx])
                    for g in range(F // 16):
                        sl = pl.ds(g * 16, 16)
                        rows[r, sl] = rows[r, sl] * w
                return carry3

            lax.fori_loop(0, CHUNK // 4, row_body, 0)
            pltpu.sync_copy(rows, acc.at[dstv2.at[j]], add=True)

        pltpu.async_copy(tab_hbm.at[srcv2.at[0]], rows_a, sem_a)

        def pair_body(jj, carry2):
            j = jj * 2
            pltpu.async_copy(tab_hbm.at[srcv2.at[j + 1]], rows_b, sem_b)
            pltpu.make_async_copy(tab_hbm.at[srcv2.at[j]], rows_a, sem_a).wait()
            process(j, rows_a)

            @pl.when(jj + 1 < G // 2)
            def _prefetch():
                pltpu.async_copy(tab_hbm.at[srcv2.at[j + 2]], rows_a, sem_a)

            pltpu.make_async_copy(tab_hbm.at[srcv2.at[j + 1]], rows_b, sem_b).wait()
            process(j + 1, rows_b)
            return carry2

        lax.fori_loop(0, G // 2, pair_body, 0)
        return carry

    lax.fori_loop(0, my_cpw // G, group_body, 0)
    plsc.subcore_barrier()
    pltpu.sync_copy(acc.at[pl.ds(s * NPS, NPS)],
                    s_out.at[c, pl.ds(s * NPS, NPS)])


_BLK = NPS


def _dis_body(degp_ref, dis_ref):
    deg = degp_ref[0:1, :] + degp_ref[1:2, :]
    dis_ref[...] = jnp.where(deg > 0, lax.rsqrt(deg), 0.0)


_dis_tc = pl.pallas_call(
    _dis_body,
    out_shape=jax.ShapeDtypeStruct((1, NP), jnp.float32),
)


def _mid_body(s1_ref, tx1_ref):
    tx1_ref[...] = s1_ref[0] + s1_ref[1]


_mid = pl.pallas_call(
    _mid_body,
    grid=(NP // _BLK,),
    in_specs=[pl.BlockSpec((2, _BLK, F), lambda i: (0, i, 0))],
    out_specs=pl.BlockSpec((_BLK, F), lambda i: (i, 0)),
    out_shape=jax.ShapeDtypeStruct((NP, F), jnp.float32),
)


def _final_body(x_ref, tx1_ref, s2_ref, wz_ref, wh_ref, bz_ref,
                bh_ref, w1_ref, b1_ref, w2_ref, b2_ref, w3_ref, b3_ref,
                out_ref):
    x = x_ref[...]
    tx1 = tx1_ref[...]
    tx2 = 2.0 * (s2_ref[0] + s2_ref[1]) - x
    xcat = jnp.concatenate([x, tx1, tx2], axis=1)
    cz = jnp.dot(xcat, wz_ref[...], preferred_element_type=jnp.float32) + bz_ref[...]
    ch = jnp.dot(xcat, wh_ref[...], preferred_element_type=jnp.float32) + bh_ref[...]
    z = jax.nn.sigmoid(cz)
    h = (1.0 - z) * jnp.tanh(ch)
    h = jnp.maximum(h, 0.0)
    h = jnp.maximum(jnp.dot(h, w1_ref[...], preferred_element_type=jnp.float32) + b1_ref[...], 0.0)
    h = jnp.maximum(jnp.dot(h, w2_ref[...], preferred_element_type=jnp.float32) + b2_ref[...], 0.0)
    lg = jnp.dot(h, w3_ref[...], preferred_element_type=jnp.float32) + b3_ref[...]
    col = lax.broadcasted_iota(jnp.int32, lg.shape, 1)
    valid = col < NUM_CLASSES
    m = jnp.max(jnp.where(valid, lg, -1e30), axis=1, keepdims=True)
    ssum = jnp.sum(jnp.where(valid, jnp.exp(lg - m), 0.0), axis=1, keepdims=True)
    res = lg - m - jnp.log(ssum)
    out_ref[...] = res[:, :NUM_CLASSES]


def _full(shape):
    return pl.BlockSpec(shape, lambda i: (0,) * len(shape))


_final = pl.pallas_call(
    _final_body,
    grid=(NP // _BLK,),
    in_specs=[
        pl.BlockSpec((_BLK, F), lambda i: (i, 0)),
        pl.BlockSpec((_BLK, F), lambda i: (i, 0)),
        pl.BlockSpec((2, _BLK, F), lambda i: (0, i, 0)),
        _full((3 * F, F)),
        _full((3 * F, F)),
        _full((1, F)),
        _full((1, F)),
        _full((F, 32)),
        _full((1, 32)),
        _full((32, 16)),
        _full((1, 16)),
        _full((16, F)),
        _full((1, F)),
    ],
    out_specs=pl.BlockSpec((_BLK, NUM_CLASSES), lambda i: (i, 0)),
    out_shape=jax.ShapeDtypeStruct((NP, NUM_CLASSES), jnp.float32),
)


def kernel(x, edge_index, edge_weight, Wxz, bxz, Whz, bhz, Wxr, bxr, Whr, bhr,
           Wxh, bxh, Whh, bhh, W1, b1, W2, b2, W3, b3):
    pad = E_PAD - E
    src1 = jnp.concatenate([edge_index[0], jnp.zeros((pad,), jnp.int32)])
    dst1 = jnp.concatenate([edge_index[1], jnp.zeros((pad,), jnp.int32)])
    ew1 = jnp.concatenate([edge_weight, jnp.zeros((pad,), jnp.float32)])
    src2 = src1.reshape(RE, CHUNK)
    dst2 = dst1.reshape(RE, CHUNK)
    z1 = jnp.zeros((NP,), jnp.float32)
    znf = jnp.zeros((NP, F), jnp.float32)
    xp = jnp.concatenate([x, jnp.zeros((NP - N, F), jnp.float32)])

    degp = _deg(src1, dst1, ew1, src2, z1)
    dis = _dis_tc(degp.reshape(2, NP)).reshape(NP)
    s1p = _row_pass(xp, src1, dst1, ew1, src2, dst2, dis, znf)
    tx1 = _mid(s1p)
    s2p = _row_pass(tx1, src1, dst1, ew1, src2, dst2, dis, znf)

    bz = (bxz + bhz).reshape(1, F)
    bh = (bxh + bhh).reshape(1, F)
    wz = Wxz.reshape(3 * F, F)
    wh = Wxh.reshape(3 * F, F)
    w3p = jnp.zeros((16, F), jnp.float32).at[:, :NUM_CLASSES].set(W3)
    b3p = jnp.zeros((1, F), jnp.float32).at[0, :NUM_CLASSES].set(b3)

    out = _final(xp, tx1, s2p, wz, wh, bz, bh,
                 W1, b1.reshape(1, 32), W2, b2.reshape(1, 16), w3p, b3p)
    return out[:N]
